# baseline (speedup 1.0000x reference)
import jax
import jax.numpy as jnp
from jax import lax
from jax.experimental import pallas as pl
from jax.experimental.pallas import tpu as pltpu
from jax.experimental.pallas import tpu_sc as plsc

BINS_ = 10
BLOCK_ROWS = 2048
SC_ROWS = 32768
NC, NS, L = 2, 16, 16
NW = NC * NS
CHUNK = 32


def _sc_body(y_flat, t_hbm, sacc_hbm, xt_hbm,
             rows_v, lab_v, idx_v, xt_v, acc_v, sem):
    wid = lax.axis_index("s") * NC + lax.axis_index("c")
    rows_per_w = SC_ROWS // NW
    nchunks = rows_per_w // CHUNK
    base_row = wid * rows_per_w
    lane = lax.iota(jnp.int32, L)
    tail_mask = lane < 8
    zero16 = jnp.zeros((L,), jnp.float32)

    def chunk_body(ck, _):
        row0 = base_row + ck * CHUNK
        pltpu.sync_copy(y_flat.at[pl.ds(row0 * 1000, CHUNK * 1000)],
                        rows_v.at[pl.ds(0, CHUNK * 1000)])
        pltpu.sync_copy(t_hbm.at[pl.ds(row0, CHUNK)], lab_v)

        def idx_body(g, _):
            lab16 = lab_v[pl.ds(g * L, L)]
            idx_v[pl.ds(g * L, L)] = (row0 + g * L + lane) * 1000 + lab16
            return 0

        lax.fori_loop(0, CHUNK // L, idx_body, 0)
        cp = pltpu.make_async_copy(y_flat.at[idx_v], xt_v, sem)
        cp.start()

        def row_body(r, _):
            roff = r * 1000

            def vec_body(j, acc):
                return acc + jnp.exp(rows_v[pl.ds(roff + j * L, L)])

            acc = lax.fori_loop(0, 62, vec_body, zero16)
            tail = rows_v[pl.ds(roff + 992, L)]
            acc = acc + jnp.exp(jnp.where(tail_mask, tail, -1e30))
            acc_v[pl.ds(r * L, L)] = acc
            return 0

        lax.fori_loop(0, CHUNK, row_body, 0)
        cp.wait()
        pltpu.sync_copy(acc_v, sacc_hbm.at[pl.ds(row0 * L, CHUNK * L)])
        pltpu.sync_copy(xt_v, xt_hbm.at[pl.ds(row0, CHUNK)])
        return 0

    lax.fori_loop(0, nchunks, chunk_body, 0)


def _tc_part(x, labels):
    b, c = x.shape
    e = jnp.exp(x)
    s = jnp.sum(e, axis=1, keepdims=True)
    cols = jax.lax.broadcasted_iota(jnp.int32, (b, c), 1)
    et = jnp.sum(jnp.where(cols == labels, e, 0.0), axis=1, keepdims=True)
    p = et / s
    bin_raw = jnp.floor((1.0 - p) * BINS_).astype(jnp.int32)
    sel = (bin_raw >= 0) & (bin_raw < BINS_)
    logp = jnp.log(p)
    binid = jax.lax.broadcasted_iota(jnp.int32, (b, BINS_), 1)
    m = (binid == bin_raw) & sel
    cnt = jnp.sum(m.astype(jnp.float32), axis=0, keepdims=True)
    slog = jnp.sum(jnp.where(m, logp, 0.0), axis=0, keepdims=True)
    return cnt, slog


def _tc_pass1(x0_ref, x1_ref, t0_ref, t1_ref, cnt_ref, slog_ref):
    c0, s0 = _tc_part(x0_ref[...], t0_ref[...])
    c1, s1 = _tc_part(x1_ref[...], t1_ref[...])
    cnt_ref[0] = c0 + c1
    slog_ref[0] = s0 + s1


def _finalize(sacc_ref, xt_ref, cnt_ref, slog_ref, out_ref):
    s = jnp.sum(sacc_ref[...], axis=2)
    xt = xt_ref[...]
    p = jnp.exp(xt) / s
    logp = xt - jnp.log(s)
    bin_raw = jnp.floor((1.0 - p) * BINS_).astype(jnp.int32)
    sel = (bin_raw >= 0) & (bin_raw < BINS_)
    cparts = []
    sparts = []
    for b in range(BINS_):
        m = (bin_raw == b) & sel
        cparts.append(jnp.sum(m.astype(jnp.float32), keepdims=True)
                      .reshape(1, 1))
        sparts.append(jnp.sum(jnp.where(m, logp, 0.0), keepdims=True)
                      .reshape(1, 1))
    counts = jnp.concatenate(cparts, axis=1)
    slog = jnp.concatenate(sparts, axis=1)
    counts = counts + jnp.sum(cnt_ref[...], axis=0)
    slog = slog + jnp.sum(slog_ref[...], axis=0)
    nonempty = counts > 0
    n = jnp.sum(nonempty.astype(jnp.float32), keepdims=True)
    per_bin = jnp.where(nonempty, slog / jnp.maximum(counts, 1.0), 0.0)
    out_ref[...] = -jnp.sum(per_bin, keepdims=True) / jnp.maximum(n, 1.0)


def kernel(y_pred, y_true):
    n, c = y_pred.shape
    tc_rows = n - SC_ROWS
    nsteps = tc_rows // (BLOCK_ROWS * 2)
    off = SC_ROWS // BLOCK_ROWS
    t2 = y_true.reshape(n, 1)

    sc_kernel = pl.kernel(
        _sc_body,
        out_type=[
            jax.ShapeDtypeStruct((SC_ROWS * L,), jnp.float32),
            jax.ShapeDtypeStruct((SC_ROWS,), jnp.float32),
        ],
        mesh=plsc.VectorSubcoreMesh(core_axis_name="c", subcore_axis_name="s"),
        scratch_types=[
            pltpu.VMEM((CHUNK * 1000 + L,), jnp.float32),
            pltpu.VMEM((CHUNK,), jnp.int32),
            pltpu.VMEM((CHUNK,), jnp.int32),
            pltpu.VMEM((CHUNK,), jnp.float32),
            pltpu.VMEM((CHUNK * L,), jnp.float32),
            pltpu.SemaphoreType.DMA,
        ],
    )
    sacc, xt = sc_kernel(y_pred.reshape(-1), y_true)

    cnt, slog = pl.pallas_call(
        _tc_pass1,
        grid=(nsteps,),
        in_specs=[
            pl.BlockSpec((BLOCK_ROWS, c), lambda i: (i + off, 0)),
            pl.BlockSpec((BLOCK_ROWS, c), lambda i: (i + off + nsteps, 0)),
            pl.BlockSpec((BLOCK_ROWS, 1), lambda i: (i + off, 0)),
            pl.BlockSpec((BLOCK_ROWS, 1), lambda i: (i + off + nsteps, 0)),
        ],
        out_specs=[
            pl.BlockSpec((1, 1, BINS_), lambda i: (i, 0, 0)),
            pl.BlockSpec((1, 1, BINS_), lambda i: (i, 0, 0)),
        ],
        out_shape=[
            jax.ShapeDtypeStruct((nsteps, 1, BINS_), jnp.float32),
            jax.ShapeDtypeStruct((nsteps, 1, BINS_), jnp.float32),
        ],
        compiler_params=pltpu.CompilerParams(
            dimension_semantics=("parallel",)),
    )(y_pred, y_pred, t2, t2)

    rblk = SC_ROWS // 128
    sacc3 = sacc.reshape(rblk, 128, L)
    xt2 = xt.reshape(rblk, 128)
    out = pl.pallas_call(
        _finalize,
        in_specs=[
            pl.BlockSpec((rblk, 128, L), lambda: (0, 0, 0)),
            pl.BlockSpec((rblk, 128), lambda: (0, 0)),
            pl.BlockSpec((nsteps, 1, BINS_), lambda: (0, 0, 0)),
            pl.BlockSpec((nsteps, 1, BINS_), lambda: (0, 0, 0)),
        ],
        out_specs=pl.BlockSpec((1, 1), lambda: (0, 0)),
        out_shape=jax.ShapeDtypeStruct((1, 1), jnp.float32),
    )(sacc3, xt2, cnt, slog)
    return out[0, 0]


# speedup vs baseline: 1.2300x; 1.2300x over previous
import jax
import jax.numpy as jnp
from jax import lax
from jax.experimental import pallas as pl
from jax.experimental.pallas import tpu as pltpu
from jax.experimental.pallas import tpu_sc as plsc

BINS_ = 10
BLOCK_ROWS = 2048
SC_ROWS = 32768
NC, NS, L = 2, 16, 16
NW = NC * NS
CHUNK = 32


def _sc_body(y_flat, t_hbm, sacc_hbm, xt_hbm,
             rows_v, lab_v, idx_v, xt_v, acc_v, sem):
    wid = lax.axis_index("s") * NC + lax.axis_index("c")
    rows_per_w = SC_ROWS // NW
    nchunks = rows_per_w // CHUNK
    base_row = wid * rows_per_w
    lane = lax.iota(jnp.int32, L)
    tail_mask = lane < 8
    zero16 = jnp.zeros((L,), jnp.float32)

    def chunk_body(ck, _):
        row0 = base_row + ck * CHUNK
        pltpu.sync_copy(y_flat.at[pl.ds(row0 * 1000, CHUNK * 1000)],
                        rows_v.at[pl.ds(0, CHUNK * 1000)])
        pltpu.sync_copy(t_hbm.at[pl.ds(row0, CHUNK)], lab_v)

        def idx_body(g, _):
            lab16 = lab_v[pl.ds(g * L, L)]
            idx_v[pl.ds(g * L, L)] = (row0 + g * L + lane) * 1000 + lab16
            return 0

        lax.fori_loop(0, CHUNK // L, idx_body, 0)
        cp = pltpu.make_async_copy(y_flat.at[idx_v], xt_v, sem)
        cp.start()

        def row_body(r, _):
            roff = r * 1000
            acc = zero16
            for j in range(62):
                acc = acc + jnp.exp(rows_v[pl.ds(roff + j * L, L)])
            tail = rows_v[pl.ds(roff + 992, L)]
            acc = acc + jnp.exp(jnp.where(tail_mask, tail, -1e30))
            acc_v[pl.ds(r * L, L)] = acc
            return 0

        lax.fori_loop(0, CHUNK, row_body, 0)
        cp.wait()
        pltpu.sync_copy(acc_v, sacc_hbm.at[pl.ds(row0 * L, CHUNK * L)])
        pltpu.sync_copy(xt_v, xt_hbm.at[pl.ds(row0, CHUNK)])
        return 0

    lax.fori_loop(0, nchunks, chunk_body, 0)


def _tc_part(x, labels):
    b, c = x.shape
    e = jnp.exp(x)
    s = jnp.sum(e, axis=1, keepdims=True)
    cols = jax.lax.broadcasted_iota(jnp.int32, (b, c), 1)
    et = jnp.sum(jnp.where(cols == labels, e, 0.0), axis=1, keepdims=True)
    p = et / s
    bin_raw = jnp.floor((1.0 - p) * BINS_).astype(jnp.int32)
    sel = (bin_raw >= 0) & (bin_raw < BINS_)
    logp = jnp.log(p)
    binid = jax.lax.broadcasted_iota(jnp.int32, (b, BINS_), 1)
    m = (binid == bin_raw) & sel
    cnt = jnp.sum(m.astype(jnp.float32), axis=0, keepdims=True)
    slog = jnp.sum(jnp.where(m, logp, 0.0), axis=0, keepdims=True)
    return cnt, slog


def _tc_pass1(x0_ref, x1_ref, t0_ref, t1_ref, cnt_ref, slog_ref):
    c0, s0 = _tc_part(x0_ref[...], t0_ref[...])
    c1, s1 = _tc_part(x1_ref[...], t1_ref[...])
    cnt_ref[0] = c0 + c1
    slog_ref[0] = s0 + s1


def _finalize(sacc_ref, xt_ref, cnt_ref, slog_ref, out_ref):
    s = jnp.sum(sacc_ref[...], axis=2)
    xt = xt_ref[...]
    p = jnp.exp(xt) / s
    logp = xt - jnp.log(s)
    bin_raw = jnp.floor((1.0 - p) * BINS_).astype(jnp.int32)
    sel = (bin_raw >= 0) & (bin_raw < BINS_)
    cparts = []
    sparts = []
    for b in range(BINS_):
        m = (bin_raw == b) & sel
        cparts.append(jnp.sum(m.astype(jnp.float32), keepdims=True)
                      .reshape(1, 1))
        sparts.append(jnp.sum(jnp.where(m, logp, 0.0), keepdims=True)
                      .reshape(1, 1))
    counts = jnp.concatenate(cparts, axis=1)
    slog = jnp.concatenate(sparts, axis=1)
    counts = counts + jnp.sum(cnt_ref[...], axis=0)
    slog = slog + jnp.sum(slog_ref[...], axis=0)
    nonempty = counts > 0
    n = jnp.sum(nonempty.astype(jnp.float32), keepdims=True)
    per_bin = jnp.where(nonempty, slog / jnp.maximum(counts, 1.0), 0.0)
    out_ref[...] = -jnp.sum(per_bin, keepdims=True) / jnp.maximum(n, 1.0)


def kernel(y_pred, y_true):
    n, c = y_pred.shape
    tc_rows = n - SC_ROWS
    nsteps = tc_rows // (BLOCK_ROWS * 2)
    off = SC_ROWS // BLOCK_ROWS
    t2 = y_true.reshape(n, 1)

    sc_kernel = pl.kernel(
        _sc_body,
        out_type=[
            jax.ShapeDtypeStruct((SC_ROWS * L,), jnp.float32),
            jax.ShapeDtypeStruct((SC_ROWS,), jnp.float32),
        ],
        mesh=plsc.VectorSubcoreMesh(core_axis_name="c", subcore_axis_name="s"),
        scratch_types=[
            pltpu.VMEM((CHUNK * 1000 + L,), jnp.float32),
            pltpu.VMEM((CHUNK,), jnp.int32),
            pltpu.VMEM((CHUNK,), jnp.int32),
            pltpu.VMEM((CHUNK,), jnp.float32),
            pltpu.VMEM((CHUNK * L,), jnp.float32),
            pltpu.SemaphoreType.DMA,
        ],
    )
    cnt, slog = pl.pallas_call(
        _tc_pass1,
        grid=(nsteps,),
        in_specs=[
            pl.BlockSpec((BLOCK_ROWS, c), lambda i: (i + off, 0)),
            pl.BlockSpec((BLOCK_ROWS, c), lambda i: (i + off + nsteps, 0)),
            pl.BlockSpec((BLOCK_ROWS, 1), lambda i: (i + off, 0)),
            pl.BlockSpec((BLOCK_ROWS, 1), lambda i: (i + off + nsteps, 0)),
        ],
        out_specs=[
            pl.BlockSpec((1, 1, BINS_), lambda i: (i, 0, 0)),
            pl.BlockSpec((1, 1, BINS_), lambda i: (i, 0, 0)),
        ],
        out_shape=[
            jax.ShapeDtypeStruct((nsteps, 1, BINS_), jnp.float32),
            jax.ShapeDtypeStruct((nsteps, 1, BINS_), jnp.float32),
        ],
        compiler_params=pltpu.CompilerParams(
            dimension_semantics=("parallel",)),
    )(y_pred, y_pred, t2, t2)

    sacc, xt = sc_kernel(y_pred.reshape(-1), y_true)

    rblk = SC_ROWS // 128
    sacc3 = sacc.reshape(rblk, 128, L)
    xt2 = xt.reshape(rblk, 128)
    out = pl.pallas_call(
        _finalize,
        in_specs=[
            pl.BlockSpec((rblk, 128, L), lambda: (0, 0, 0)),
            pl.BlockSpec((rblk, 128), lambda: (0, 0)),
            pl.BlockSpec((nsteps, 1, BINS_), lambda: (0, 0, 0)),
            pl.BlockSpec((nsteps, 1, BINS_), lambda: (0, 0, 0)),
        ],
        out_specs=pl.BlockSpec((1, 1), lambda: (0, 0)),
        out_shape=jax.ShapeDtypeStruct((1, 1), jnp.float32),
    )(sacc3, xt2, cnt, slog)
    return out[0, 0]


# speedup vs baseline: 1.2301x; 1.0001x over previous
import jax
import jax.numpy as jnp
from jax import lax
from jax.experimental import pallas as pl
from jax.experimental.pallas import tpu as pltpu
from jax.experimental.pallas import tpu_sc as plsc

BINS_ = 10
BLOCK_ROWS = 2048
SC_ROWS = 32768
NC, NS, L = 2, 16, 16
NW = NC * NS
CHUNK = 32


def _sc_body(y_flat, t_hbm, sacc_hbm, xt_hbm,
             rows_v, lab_v, idx_v, xt_v, acc_v, sem):
    wid = lax.axis_index("s") * NC + lax.axis_index("c")
    rows_per_w = SC_ROWS // NW
    nchunks = rows_per_w // CHUNK
    base_row = wid * rows_per_w
    lane = lax.iota(jnp.int32, L)
    tail_mask = lane < 8
    zero16 = jnp.zeros((L,), jnp.float32)

    def chunk_body(ck, _):
        row0 = base_row + ck * CHUNK
        pltpu.sync_copy(y_flat.at[pl.ds(row0 * 1000, CHUNK * 1000)],
                        rows_v.at[pl.ds(0, CHUNK * 1000)])
        pltpu.sync_copy(t_hbm.at[pl.ds(row0, CHUNK)], lab_v)

        def idx_body(g, _):
            lab16 = lab_v[pl.ds(g * L, L)]
            idx_v[pl.ds(g * L, L)] = (row0 + g * L + lane) * 1000 + lab16
            return 0

        lax.fori_loop(0, CHUNK // L, idx_body, 0)
        cp = pltpu.make_async_copy(y_flat.at[idx_v], xt_v, sem)
        cp.start()

        def row_body(r, _):
            roff = r * 1000
            acc = zero16
            for j in range(62):
                acc = acc + jnp.exp(rows_v[pl.ds(roff + j * L, L)])
            tail = rows_v[pl.ds(roff + 992, L)]
            acc = acc + jnp.exp(jnp.where(tail_mask, tail, -1e30))
            acc_v[pl.ds(r * L, L)] = acc
            return 0

        lax.fori_loop(0, CHUNK, row_body, 0)
        cp.wait()
        pltpu.sync_copy(acc_v, sacc_hbm.at[pl.ds(row0 * L, CHUNK * L)])
        pltpu.sync_copy(xt_v, xt_hbm.at[pl.ds(row0, CHUNK)])
        return 0

    lax.fori_loop(0, nchunks, chunk_body, 0)


def _tc_part(x, labels):
    b, c = x.shape
    e = jnp.exp(x)
    s = jnp.sum(e, axis=1, keepdims=True)
    cols = jax.lax.broadcasted_iota(jnp.int32, (b, c), 1)
    et = jnp.sum(jnp.where(cols == labels, e, 0.0), axis=1, keepdims=True)
    p = et / s
    bin_raw = jnp.floor((1.0 - p) * BINS_).astype(jnp.int32)
    sel = (bin_raw >= 0) & (bin_raw < BINS_)
    logp = jnp.log(p)
    binid = jax.lax.broadcasted_iota(jnp.int32, (b, BINS_), 1)
    m = (binid == bin_raw) & sel
    cnt = jnp.sum(m.astype(jnp.float32), axis=0, keepdims=True)
    slog = jnp.sum(jnp.where(m, logp, 0.0), axis=0, keepdims=True)
    return cnt, slog


def _tc_pass1(x0_ref, x1_ref, t0_ref, t1_ref, cnt_ref, slog_ref):
    c0, s0 = _tc_part(x0_ref[...], t0_ref[...])
    c1, s1 = _tc_part(x1_ref[...], t1_ref[...])
    cnt_ref[0] = c0 + c1
    slog_ref[0] = s0 + s1


def _finalize(sacc_ref, xt_ref, cnt_ref, slog_ref, out_ref):
    s = jnp.sum(sacc_ref[...], axis=2)
    xt = xt_ref[...]
    p = jnp.exp(xt) / s
    logp = xt - jnp.log(s)
    bin_raw = jnp.floor((1.0 - p) * BINS_).astype(jnp.int32)
    sel = (bin_raw >= 0) & (bin_raw < BINS_)
    cparts = []
    sparts = []
    for b in range(BINS_):
        m = (bin_raw == b) & sel
        cparts.append(jnp.sum(m.astype(jnp.float32), keepdims=True)
                      .reshape(1, 1))
        sparts.append(jnp.sum(jnp.where(m, logp, 0.0), keepdims=True)
                      .reshape(1, 1))
    counts = jnp.concatenate(cparts, axis=1)
    slog = jnp.concatenate(sparts, axis=1)
    counts = counts + jnp.sum(cnt_ref[...], axis=0)
    slog = slog + jnp.sum(slog_ref[...], axis=0)
    nonempty = counts > 0
    n = jnp.sum(nonempty.astype(jnp.float32), keepdims=True)
    per_bin = jnp.where(nonempty, slog / jnp.maximum(counts, 1.0), 0.0)
    out_ref[...] = -jnp.sum(per_bin, keepdims=True) / jnp.maximum(n, 1.0)


def kernel(y_pred, y_true):
    n, c = y_pred.shape
    tc_rows = n - SC_ROWS
    nsteps = tc_rows // (BLOCK_ROWS * 2)
    off = SC_ROWS // BLOCK_ROWS
    t2 = y_true.reshape(n, 1)

    sc_kernel = pl.kernel(
        _sc_body,
        out_type=[
            jax.ShapeDtypeStruct((SC_ROWS * L,), jnp.float32),
            jax.ShapeDtypeStruct((SC_ROWS,), jnp.float32),
        ],
        mesh=plsc.VectorSubcoreMesh(core_axis_name="c", subcore_axis_name="s"),
        scratch_types=[
            pltpu.VMEM((CHUNK * 1000 + L,), jnp.float32),
            pltpu.VMEM((CHUNK,), jnp.int32),
            pltpu.VMEM((CHUNK,), jnp.int32),
            pltpu.VMEM((CHUNK,), jnp.float32),
            pltpu.VMEM((CHUNK * L,), jnp.float32),
            pltpu.SemaphoreType.DMA,
        ],
        cost_estimate=pl.CostEstimate(
            flops=SC_ROWS * 1000 * 2,
            transcendentals=SC_ROWS * 1000,
            bytes_accessed=SC_ROWS * 1000 * 4,
        ),
    )
    sacc, xt = sc_kernel(y_pred.reshape(-1), y_true)

    cnt, slog = pl.pallas_call(
        _tc_pass1,
        grid=(nsteps,),
        in_specs=[
            pl.BlockSpec((BLOCK_ROWS, c), lambda i: (i + off, 0)),
            pl.BlockSpec((BLOCK_ROWS, c), lambda i: (i + off + nsteps, 0)),
            pl.BlockSpec((BLOCK_ROWS, 1), lambda i: (i + off, 0)),
            pl.BlockSpec((BLOCK_ROWS, 1), lambda i: (i + off + nsteps, 0)),
        ],
        out_specs=[
            pl.BlockSpec((1, 1, BINS_), lambda i: (i, 0, 0)),
            pl.BlockSpec((1, 1, BINS_), lambda i: (i, 0, 0)),
        ],
        out_shape=[
            jax.ShapeDtypeStruct((nsteps, 1, BINS_), jnp.float32),
            jax.ShapeDtypeStruct((nsteps, 1, BINS_), jnp.float32),
        ],
        compiler_params=pltpu.CompilerParams(
            dimension_semantics=("parallel",)),
    )(y_pred, y_pred, t2, t2)

    rblk = SC_ROWS // 128
    sacc3 = sacc.reshape(rblk, 128, L)
    xt2 = xt.reshape(rblk, 128)
    out = pl.pallas_call(
        _finalize,
        in_specs=[
            pl.BlockSpec((rblk, 128, L), lambda: (0, 0, 0)),
            pl.BlockSpec((rblk, 128), lambda: (0, 0)),
            pl.BlockSpec((nsteps, 1, BINS_), lambda: (0, 0, 0)),
            pl.BlockSpec((nsteps, 1, BINS_), lambda: (0, 0, 0)),
        ],
        out_specs=pl.BlockSpec((1, 1), lambda: (0, 0)),
        out_shape=jax.ShapeDtypeStruct((1, 1), jnp.float32),
    )(sacc3, xt2, cnt, slog)
    return out[0, 0]


# speedup vs baseline: 9.3930x; 7.6358x over previous
import functools

import jax
import jax.numpy as jnp
from jax.experimental import pallas as pl
from jax.experimental.pallas import tpu as pltpu

BINS_ = 10
BC = 2048


def _part(x, labels):
    s = jnp.sum(jnp.exp(x), axis=0, keepdims=True)
    rows = jax.lax.broadcasted_iota(jnp.int32, x.shape, 0)
    xt = jnp.sum(jnp.where(rows == labels, x, 0.0), axis=0, keepdims=True)
    p = jnp.exp(xt) / s
    bin_raw = jnp.floor((1.0 - p) * BINS_).astype(jnp.int32)
    sel = (bin_raw >= 0) & (bin_raw < BINS_)
    logp = xt - jnp.log(s)
    return bin_raw, sel, logp


def _t_kernel(x0_ref, x1_ref, t0_ref, t1_ref, out_ref, acc_ref, *, nsteps):
    i = pl.program_id(0)

    @pl.when(i == 0)
    def _init():
        acc_ref[...] = jnp.zeros_like(acc_ref)

    b0, s0, l0 = _part(x0_ref[...], t0_ref[...])
    b1, s1, l1 = _part(x1_ref[...], t1_ref[...])
    cnts = []
    slogs = []
    for b in range(BINS_):
        m0 = (b0 == b) & s0
        m1 = (b1 == b) & s1
        c = (jnp.sum(m0.astype(jnp.float32), keepdims=True)
             + jnp.sum(m1.astype(jnp.float32), keepdims=True))
        sl = (jnp.sum(jnp.where(m0, l0, 0.0), keepdims=True)
              + jnp.sum(jnp.where(m1, l1, 0.0), keepdims=True))
        cnts.append(c.reshape(1, 1))
        slogs.append(sl.reshape(1, 1))
    acc_ref[0:1, :] += jnp.concatenate(cnts, axis=1)
    acc_ref[1:2, :] += jnp.concatenate(slogs, axis=1)

    @pl.when(i == nsteps - 1)
    def _fin():
        counts = acc_ref[0:1, :]
        slog = acc_ref[1:2, :]
        nonempty = counts > 0
        n = jnp.sum(nonempty.astype(jnp.float32), keepdims=True)
        per_bin = jnp.where(nonempty, slog / jnp.maximum(counts, 1.0), 0.0)
        out_ref[...] = (-jnp.sum(per_bin, keepdims=True)
                        / jnp.maximum(n, 1.0))


def kernel(y_pred, y_true):
    n, c = y_pred.shape
    xT = y_pred.T
    tl = y_true.reshape(1, n)
    nsteps = n // (BC * 2)
    out = pl.pallas_call(
        functools.partial(_t_kernel, nsteps=nsteps),
        grid=(nsteps,),
        in_specs=[
            pl.BlockSpec((c, BC), lambda i: (0, i)),
            pl.BlockSpec((c, BC), lambda i: (0, i + nsteps)),
            pl.BlockSpec((1, BC), lambda i: (0, i)),
            pl.BlockSpec((1, BC), lambda i: (0, i + nsteps)),
        ],
        out_specs=pl.BlockSpec((1, 1), lambda i: (0, 0)),
        out_shape=jax.ShapeDtypeStruct((1, 1), jnp.float32),
        scratch_shapes=[pltpu.VMEM((2, BINS_), jnp.float32)],
    )(xT, xT, tl, tl)
    return out[0, 0]
